# baseline (speedup 1.0000x reference)
import functools

import jax
import jax.numpy as jnp
from jax.experimental import pallas as pl
from jax.experimental.pallas import tpu as pltpu

H, E, K = 1024, 8, 2
N = 4096
BLK_R = 256
G = (N * K) // BLK_R + (E - 1)
P = G * BLK_R


def _router_body(x_ref, rw_ref, i1_ref, i2_ref, w1_ref, w2_ref, b_ref):
    x = x_ref[...]
    logits = jnp.dot(x, rw_ref[...].T, preferred_element_type=jnp.float32)
    m = jnp.max(logits, axis=-1, keepdims=True)
    p = jnp.exp(logits - m)
    probs = p / jnp.sum(p, axis=-1, keepdims=True)
    iota = jax.lax.broadcasted_iota(jnp.int32, probs.shape, 1)
    m1 = jnp.max(probs, axis=-1, keepdims=True)
    i1 = jnp.min(jnp.where(probs == m1, iota, E), axis=-1, keepdims=True)
    probs2 = jnp.where(iota == i1, -jnp.inf, probs)
    m2 = jnp.max(probs2, axis=-1, keepdims=True)
    i2 = jnp.min(jnp.where(probs2 == m2, iota, E), axis=-1, keepdims=True)
    denom = m1 + m2 + 1e-6
    w1 = m1 / denom
    w2 = m2 / denom
    i1_ref[...] = i1
    i2_ref[...] = i2
    w1_ref[...] = w1
    w2_ref[...] = w2
    b_ref[...] = w1 * m1 + w2 * m2


def _router(x, router_w):
    blk = 2048
    o = jax.ShapeDtypeStruct((N, 1), jnp.float32)
    oi = jax.ShapeDtypeStruct((N, 1), jnp.int32)
    spec = pl.BlockSpec((blk, 1), lambda i: (i, 0))
    return pl.pallas_call(
        _router_body,
        grid=(N // blk,),
        in_specs=[
            pl.BlockSpec((blk, H), lambda i: (i, 0)),
            pl.BlockSpec((E, H), lambda i: (0, 0)),
        ],
        out_specs=[spec] * 5,
        out_shape=[oi, oi, o, o, o],
    )(x, router_w)


def _gmm_body(eb_ref, x_ref, w_ref, wa_ref, y_ref):
    y = jnp.dot(x_ref[...], w_ref[0], preferred_element_type=jnp.float32)
    y_ref[...] = (y * wa_ref[...]).astype(jnp.bfloat16)


def _gmm(eb, x_sorted, ws, wa_pad):
    grid_spec = pltpu.PrefetchScalarGridSpec(
        num_scalar_prefetch=1,
        grid=(G,),
        in_specs=[
            pl.BlockSpec((BLK_R, H), lambda t, eb: (t, 0)),
            pl.BlockSpec((1, H, H), lambda t, eb: (eb[t], 0, 0)),
            pl.BlockSpec((BLK_R, 1), lambda t, eb: (t, 0)),
        ],
        out_specs=pl.BlockSpec((BLK_R, H), lambda t, eb: (t, 0)),
    )
    return pl.pallas_call(
        _gmm_body,
        grid_spec=grid_spec,
        out_shape=jax.ShapeDtypeStruct((P, H), jnp.bfloat16),
    )(eb, x_sorted, ws, wa_pad)


@jax.jit
def _moe(x, router_w, ws_bf16):
    i1, i2, w1, w2, bias = _router(x, router_w)
    ea = jnp.concatenate([i1, i2]).reshape(-1)
    wa = jnp.concatenate([w1, w2]).reshape(-1)
    order = jnp.argsort(ea)
    counts = jnp.bincount(ea, length=E)
    offs = jnp.concatenate([jnp.zeros(1, jnp.int32),
                            jnp.cumsum(counts)[:-1].astype(jnp.int32)])
    nblk = -(-counts // BLK_R)
    blk_off = jnp.concatenate([jnp.zeros(1, jnp.int32),
                               jnp.cumsum(nblk)[:-1].astype(jnp.int32)])
    cum_nblk = jnp.cumsum(nblk).astype(jnp.int32)
    t_iota = jnp.arange(G, dtype=jnp.int32)
    eb = jnp.searchsorted(cum_nblk, t_iota, side='right').astype(jnp.int32)
    eb = jnp.minimum(eb, E - 1)
    p_iota = jnp.arange(P, dtype=jnp.int32)
    ep = eb[p_iota // BLK_R]
    rank = p_iota - blk_off[ep] * BLK_R
    valid = rank < counts[ep]
    j = offs[ep] + jnp.minimum(rank, jnp.maximum(counts[ep] - 1, 0))
    a = order[j]
    src_row = jnp.where(valid, a % N, 0)
    wa_pad = jnp.where(valid, wa[a], 0.0)[:, None]
    jj = jnp.arange(N * K, dtype=jnp.int32)
    pos_sorted = blk_off[ea[order]] * BLK_R + (jj - offs[ea[order]])
    dpos = jnp.zeros(N * K, jnp.int32).at[order].set(pos_sorted)
    p1, p2 = dpos[:N], dpos[N:]
    x_sorted = x.astype(jnp.bfloat16)[src_row]
    y = _gmm(eb, x_sorted, ws_bf16, wa_pad)
    out = y[p1].astype(jnp.float32) + y[p2].astype(jnp.float32) + bias
    return out


def kernel(hidden_states, router_w, expert_weights, expert_mapping):
    b, s, h = hidden_states.shape
    x = hidden_states.reshape(-1, h)
    out = _moe(x, router_w, expert_weights.astype(jnp.bfloat16))
    return out.reshape(b, s, h)


# speedup vs baseline: 1.9446x; 1.9446x over previous
import functools

import jax
import jax.numpy as jnp
from jax.experimental import pallas as pl
from jax.experimental.pallas import tpu as pltpu

H, E, K = 1024, 8, 2
N = 4096
BLK_R = 256
G = (N * K) // BLK_R + (E - 1)
P = G * BLK_R
RBLK = 256


def _router_body(x_ref, rw_ref, i1_ref, i2_ref, w1_ref, w2_ref, b_ref,
                 r1_ref, r2_ref, c_ref, base_ref):
    @pl.when(pl.program_id(0) == 0)
    def _init():
        base_ref[...] = jnp.zeros_like(base_ref)

    x = x_ref[...]
    logits = jnp.dot(x, rw_ref[...].T, preferred_element_type=jnp.float32)
    m = jnp.max(logits, axis=-1, keepdims=True)
    p = jnp.exp(logits - m)
    probs = p / jnp.sum(p, axis=-1, keepdims=True)
    iota = jax.lax.broadcasted_iota(jnp.int32, probs.shape, 1)
    m1 = jnp.max(probs, axis=-1, keepdims=True)
    i1 = jnp.min(jnp.where(probs == m1, iota, E), axis=-1, keepdims=True)
    probs2 = jnp.where(iota == i1, -jnp.inf, probs)
    m2 = jnp.max(probs2, axis=-1, keepdims=True)
    i2 = jnp.min(jnp.where(probs2 == m2, iota, E), axis=-1, keepdims=True)
    denom = m1 + m2 + 1e-6
    w1 = m1 / denom
    w2 = m2 / denom
    i1_ref[...] = i1
    i2_ref[...] = i2
    w1_ref[...] = w1
    w2_ref[...] = w2
    b_ref[...] = w1 * m1 + w2 * m2
    oh1 = (iota == i1).astype(jnp.float32)
    oh2 = (iota == i2).astype(jnp.float32)
    ri = jax.lax.broadcasted_iota(jnp.int32, (RBLK, RBLK), 0)
    ci = jax.lax.broadcasted_iota(jnp.int32, (RBLK, RBLK), 1)
    tri = (ri > ci).astype(jnp.float32)
    base = base_ref[...]
    r1 = base + jnp.dot(tri, oh1, preferred_element_type=jnp.float32)
    base = base + jnp.sum(oh1, axis=0, keepdims=True)
    r2 = base + jnp.dot(tri, oh2, preferred_element_type=jnp.float32)
    base = base + jnp.sum(oh2, axis=0, keepdims=True)
    base_ref[...] = base
    r1_ref[...] = jnp.sum(r1 * oh1, axis=-1, keepdims=True).astype(jnp.int32)
    r2_ref[...] = jnp.sum(r2 * oh2, axis=-1, keepdims=True).astype(jnp.int32)
    c_ref[...] = base


def _router(x, router_w):
    o = jax.ShapeDtypeStruct((N, 1), jnp.float32)
    oi = jax.ShapeDtypeStruct((N, 1), jnp.int32)
    spec = pl.BlockSpec((RBLK, 1), lambda i: (i, 0))
    return pl.pallas_call(
        _router_body,
        grid=(N // RBLK,),
        in_specs=[
            pl.BlockSpec((RBLK, H), lambda i: (i, 0)),
            pl.BlockSpec((E, H), lambda i: (0, 0)),
        ],
        out_specs=[spec, spec, spec, spec, spec, spec, spec,
                   pl.BlockSpec((1, E), lambda i: (0, 0))],
        out_shape=[oi, oi, o, o, o, oi, oi,
                   jax.ShapeDtypeStruct((1, E), jnp.float32)],
        scratch_shapes=[pltpu.VMEM((1, E), jnp.float32)],
    )(x, router_w)


def _gmm_body(eb_ref, x_ref, w_ref, wa_ref, y_ref):
    y = jnp.dot(x_ref[...], w_ref[0], preferred_element_type=jnp.float32)
    y_ref[...] = (y * wa_ref[...]).astype(jnp.bfloat16)


def _gmm(eb, x_sorted, ws, wa_pad):
    grid_spec = pltpu.PrefetchScalarGridSpec(
        num_scalar_prefetch=1,
        grid=(G,),
        in_specs=[
            pl.BlockSpec((BLK_R, H), lambda t, eb: (t, 0)),
            pl.BlockSpec((1, H, H), lambda t, eb: (eb[t], 0, 0)),
            pl.BlockSpec((BLK_R, 1), lambda t, eb: (t, 0)),
        ],
        out_specs=pl.BlockSpec((BLK_R, H), lambda t, eb: (t, 0)),
    )
    return pl.pallas_call(
        _gmm_body,
        grid_spec=grid_spec,
        out_shape=jax.ShapeDtypeStruct((P, H), jnp.bfloat16),
    )(eb, x_sorted, ws, wa_pad)


@jax.jit
def _moe(x, router_w, ws_bf16):
    i1, i2, w1, w2, bias, r1, r2, cnt = _router(x, router_w)
    i1, i2 = i1[:, 0], i2[:, 0]
    r1, r2 = r1[:, 0], r2[:, 0]
    counts = cnt.reshape(E).astype(jnp.int32)
    nblk = -(-counts // BLK_R)
    blk_off = jnp.concatenate([jnp.zeros(1, jnp.int32),
                               jnp.cumsum(nblk)[:-1].astype(jnp.int32)])
    cum_nblk = jnp.cumsum(nblk).astype(jnp.int32)
    t_iota = jnp.arange(G, dtype=jnp.int32)
    eb = jnp.minimum(jnp.searchsorted(cum_nblk, t_iota, side='right')
                     .astype(jnp.int32), E - 1)
    p1 = blk_off[i1] * BLK_R + r1
    p2 = blk_off[i2] * BLK_R + r2
    tok = jnp.arange(N, dtype=jnp.int32)
    src_row = jnp.zeros(P, jnp.int32).at[p1].set(tok).at[p2].set(tok)
    wa_pad = (jnp.zeros(P, jnp.float32).at[p1].set(w1[:, 0])
              .at[p2].set(w2[:, 0]))[:, None]
    x_sorted = x.astype(jnp.bfloat16)[src_row]
    y = _gmm(eb, x_sorted, ws_bf16, wa_pad)
    out = y[p1].astype(jnp.float32) + y[p2].astype(jnp.float32) + bias
    return out


def kernel(hidden_states, router_w, expert_weights, expert_mapping):
    b, s, h = hidden_states.shape
    x = hidden_states.reshape(-1, h)
    out = _moe(x, router_w, expert_weights.astype(jnp.bfloat16))
    return out.reshape(b, s, h)
